# Initial kernel scaffold; baseline (speedup 1.0000x reference)
import jax
import jax.numpy as jnp
from jax.experimental import pallas as pl


def kernel(uid_batch, mid_batch, cat_batch, mid_his_batch, cat_his_batch, mids_tri0, cats_tri0, wi_tri0, mid0_his, cat0_his, wi0_his, mids_tri1, cats_tri1, wi_tri1, mid1_his, cat1_his, wi1_his, uid_table, mid_table, cat_table, pos_table):
    raise NotImplementedError("write your pallas kernel here")


# speedup vs baseline: 3.0801x; 3.0801x over previous
import functools

import jax
import jax.numpy as jnp
from jax import lax
from jax.experimental import pallas as pl
from jax.experimental.pallas import tpu as pltpu
from jax.experimental.pallas import tpu_sc as plsc

MAXLEN = 50
EMB = 32
B = 1024
N_MID = 1000000

NC = 2
NS = 16
NW = NC * NS

CH = 960
SUB = 128

N_ITEM = 2 * B
N_HIS = 2 * B * MAXLEN
N_UB = 2 * B * MAXLEN * 9
N_CAND = 2 * B * 9

CH_HIS = 800


def _subs(ch):
    return [(o, min(SUB, ch - o)) for o in range(0, ch, SUB)]


def _gather_chunk(table, idx_v, buf, sem, ch):
    cps = [
        pltpu.async_copy(table.at[idx_v.at[pl.ds(o, sz)]],
                         buf.at[pl.ds(o, sz), :], sem)
        for (o, sz) in _subs(ch)
    ]
    for cp in cps:
        cp.wait()


def _stream_job(sidx, out, table, idx_v, buf, sem_g, sem_w, wid, per_tile,
                ch, extra=None):
    nchunks = per_tile // ch
    base0 = wid * per_tile

    def chunk_body(c, carry):
        base = base0 + c * ch
        pltpu.sync_copy(sidx.at[pl.ds(base, ch)], idx_v.at[pl.ds(0, ch)])
        _gather_chunk(table, idx_v, buf, sem_g, ch)
        w = pltpu.async_copy(buf.at[pl.ds(0, ch), :],
                             out.at[pl.ds(base, ch), :], sem_w)
        if extra is not None:
            extra(c)
        w.wait()
        return carry

    lax.fori_loop(0, nchunks, chunk_body, 0)


@functools.partial(
    pl.kernel,
    mesh=plsc.VectorSubcoreMesh(core_axis_name="c", subcore_axis_name="s"),
    compiler_params=pltpu.CompilerParams(use_tc_tiling_on_sc=False),
    out_type=[
        jax.ShapeDtypeStruct((B, EMB), jnp.float32),
        jax.ShapeDtypeStruct((N_ITEM, EMB), jnp.float32),
        jax.ShapeDtypeStruct((N_HIS, EMB), jnp.float32),
        jax.ShapeDtypeStruct((B, 2 * EMB), jnp.float32),
        jax.ShapeDtypeStruct((B, MAXLEN * 2), jnp.float32),
        jax.ShapeDtypeStruct((N_UB, EMB), jnp.float32),
        jax.ShapeDtypeStruct((N_CAND, EMB), jnp.float32),
        jax.ShapeDtypeStruct((N_UB, EMB), jnp.float32),
        jax.ShapeDtypeStruct((N_CAND, EMB), jnp.float32),
    ],
    scratch_types=[
        pltpu.VMEM((CH,), jnp.int32),
        pltpu.VMEM((CH, EMB), jnp.float32),
        pltpu.VMEM((32, 2 * EMB), jnp.float32),
        pltpu.VMEM((MAXLEN * 2,), jnp.float32),
        pltpu.SemaphoreType.DMA,
        pltpu.SemaphoreType.DMA,
    ],
)
def _tgin_sc(uid_idx, item_sidx, his_sidx, ub0_sidx, cand0_sidx,
             ub1_sidx, cand1_sidx,
             uid_table, mc_table, pos_flat,
             uid_out, item_out, his_out, sum_out, pos_out,
             ub0_out, cand0_out, ub1_out, cand1_out,
             idx_v, buf, sum_buf, pos_v, sem_g, sem_w):
    wid = lax.axis_index("s") * NC + lax.axis_index("c")

    _stream_job(uid_idx, uid_out, uid_table, idx_v, buf, sem_g, sem_w,
                wid, B // NW, B // NW)

    ubase = wid * (B // NW)
    pltpu.sync_copy(pos_flat, pos_v)
    pos_cps = [
        pltpu.async_copy(pos_v, pos_out.at[ubase + i], sem_w)
        for i in range(B // NW)
    ]
    for cp in pos_cps:
        cp.wait()

    _stream_job(item_sidx, item_out, mc_table, idx_v, buf, sem_g, sem_w,
                wid, N_ITEM // NW, N_ITEM // NW)

    def his_extra(c):
        nb = CH_HIS // (2 * MAXLEN)

        def b_body(bl, carry):
            def l_body(l, acc):
                r = bl * (2 * MAXLEN) + 2 * l
                return (acc[0] + buf[r, pl.ds(0, 16)],
                        acc[1] + buf[r, pl.ds(16, 16)],
                        acc[2] + buf[r + 1, pl.ds(0, 16)],
                        acc[3] + buf[r + 1, pl.ds(16, 16)])
            z = jnp.zeros((16,), jnp.float32)
            a0, a1, a2, a3 = lax.fori_loop(0, MAXLEN, l_body, (z, z, z, z))
            row = c * nb + bl
            sum_buf[row, pl.ds(0, 16)] = a0
            sum_buf[row, pl.ds(16, 16)] = a1
            sum_buf[row, pl.ds(32, 16)] = a2
            sum_buf[row, pl.ds(48, 16)] = a3
            return carry

        lax.fori_loop(0, nb, b_body, 0)

    _stream_job(his_sidx, his_out, mc_table, idx_v, buf, sem_g, sem_w,
                wid, N_HIS // NW, CH_HIS, extra=his_extra)
    pltpu.sync_copy(sum_buf, sum_out.at[pl.ds(wid * (B // NW), B // NW), :])

    _stream_job(ub0_sidx, ub0_out, mc_table, idx_v, buf, sem_g, sem_w,
                wid, N_UB // NW, CH)
    _stream_job(cand0_sidx, cand0_out, mc_table, idx_v, buf, sem_g, sem_w,
                wid, N_CAND // NW, N_CAND // NW)
    _stream_job(ub1_sidx, ub1_out, mc_table, idx_v, buf, sem_g, sem_w,
                wid, N_UB // NW, CH)
    _stream_job(cand1_sidx, cand1_out, mc_table, idx_v, buf, sem_g, sem_w,
                wid, N_CAND // NW, N_CAND // NW)


def _ileave(midx, cidx):
    m = midx.reshape(-1).astype(jnp.int32)
    c = cidx.reshape(-1).astype(jnp.int32) + N_MID
    return jnp.stack([m, c], axis=1).reshape(-1)


def kernel(uid_batch, mid_batch, cat_batch, mid_his_batch, cat_his_batch,
           mids_tri0, cats_tri0, wi_tri0, mid0_his, cat0_his, wi0_his,
           mids_tri1, cats_tri1, wi_tri1, mid1_his, cat1_his, wi1_his,
           uid_table, mid_table, cat_table, pos_table):
    mc_table = jnp.concatenate([mid_table, cat_table], axis=0)
    outs = _tgin_sc(
        uid_batch.astype(jnp.int32),
        _ileave(mid_batch, cat_batch),
        _ileave(mid_his_batch, cat_his_batch),
        _ileave(mid0_his, cat0_his),
        _ileave(mids_tri0, cats_tri0),
        _ileave(mid1_his, cat1_his),
        _ileave(mids_tri1, cats_tri1),
        uid_table, mc_table,
        pos_table.reshape(-1),
    )
    (uid_emb, item2, his2, his_sum, pos2,
     ub0_2, cand0_2, ub1_2, cand1_2) = outs

    return (uid_emb,
            item2.reshape(B, 2 * EMB),
            his2.reshape(B, MAXLEN, 2 * EMB),
            his_sum,
            pos2.reshape(B, MAXLEN, 2),
            ub0_2.reshape(B, MAXLEN, 9, 2 * EMB),
            wi0_his[..., None],
            cand0_2.reshape(B, 9, 2 * EMB),
            wi_tri0[..., None],
            ub1_2.reshape(B, MAXLEN, 9, 2 * EMB),
            wi1_his[..., None],
            cand1_2.reshape(B, 9, 2 * EMB),
            wi_tri1[..., None])
